# baseline (speedup 1.0000x reference)
import functools

import jax
import jax.numpy as jnp
from jax.experimental import pallas as pl
from jax.experimental.pallas import tpu as pltpu

_K = 64


def _enc_body(x_ref, we_ref, be_ref, bd_ref, pre_ref, xs_ref):
    j = pl.program_id(0)

    @pl.when(j == 0)
    def _():
        xs_ref[:] = x_ref[:] - bd_ref[:]

    acc = jnp.dot(xs_ref[:], we_ref[:], preferred_element_type=jnp.float32)
    pre_ref[:] = jnp.maximum(acc + be_ref[:], 0.0)


def _mask_body(pre_ref, z_ref, *, k, n_c):
    z = pre_ref[:]
    bits = jax.lax.bitcast_convert_type(z, jnp.int32)
    n = z.shape[1]
    r = z.shape[0]
    blk_c = n // n_c

    def count_ge(arr, cand, acc_dtype):
        cnt = jnp.zeros((r, 1), acc_dtype)
        for c in range(n_c):
            cnt += jnp.sum(
                (arr[:, c * blk_c:(c + 1) * blk_c] >= cand).astype(acc_dtype),
                axis=1, keepdims=True)
        return cnt

    hi = (bits >> 16).astype(jnp.int16)
    lo16 = jnp.zeros((r, 1), jnp.int32)
    cnt = jnp.full((r, 1), n, jnp.int32)
    for bit in range(14, -1, -1):
        cand = lo16 | (1 << bit)
        c = count_ge(hi, cand.astype(jnp.int16), jnp.int16).astype(jnp.int32)
        take = c >= k
        lo16 = jnp.where(take, cand, lo16)
        cnt = jnp.where(take, c, cnt)

    lo0 = lo16 << 16

    def cond(st):
        it, _, cnt_ = st
        return jnp.logical_and(it < 16, jnp.any(cnt_ > k))

    def body(st):
        it, lo_, cnt_ = st
        cand = lo_ | (jnp.int32(1) << (15 - it))
        c = count_ge(bits, cand, jnp.int32)
        take = c >= k
        return (it + 1, jnp.where(take, cand, lo_), jnp.where(take, c, cnt_))

    _, lo, _ = jax.lax.while_loop(cond, body, (jnp.int32(0), lo0, cnt))
    z_ref[:] = jnp.where(bits >= lo, z, 0.0).astype(jnp.bfloat16)


def _dec_body(z_ref, wd_ref, bd_ref, o_ref):
    kk = pl.program_id(0)

    @pl.when(kk == 0)
    def _():
        o_ref[:] = jnp.zeros_like(o_ref) + bd_ref[:]

    o_ref[:] += jnp.dot(z_ref[:], wd_ref[:], preferred_element_type=jnp.float32)


def kernel(x, W_enc, b_enc, W_dec, b_dec):
    b, s, d_model = x.shape
    d_sae = W_enc.shape[1]
    rows = b * s
    x_flat = x.reshape(rows, d_model)

    blk_j = min(512, d_sae)
    n_j = d_sae // blk_j

    pre = pl.pallas_call(
        _enc_body,
        grid=(n_j,),
        in_specs=[
            pl.BlockSpec((rows, d_model), lambda j: (0, 0)),
            pl.BlockSpec((d_model, blk_j), lambda j: (0, j)),
            pl.BlockSpec((1, blk_j), lambda j: (0, j)),
            pl.BlockSpec((1, d_model), lambda j: (0, 0)),
        ],
        out_specs=pl.BlockSpec((rows, blk_j), lambda j: (0, j)),
        out_shape=jax.ShapeDtypeStruct((rows, d_sae), jnp.float32),
        scratch_shapes=[pltpu.VMEM((rows, d_model), jnp.float32)],
        compiler_params=pltpu.CompilerParams(
            dimension_semantics=("arbitrary",)),
    )(x_flat, W_enc, b_enc.reshape(1, d_sae), b_dec.reshape(1, d_model))

    blk_i = min(128, rows)
    n_i = rows // blk_i
    z = pl.pallas_call(
        functools.partial(_mask_body, k=_K, n_c=4),
        grid=(n_i,),
        in_specs=[pl.BlockSpec((blk_i, d_sae), lambda i: (i, 0))],
        out_specs=pl.BlockSpec((blk_i, d_sae), lambda i: (i, 0)),
        out_shape=jax.ShapeDtypeStruct((rows, d_sae), jnp.bfloat16),
        compiler_params=pltpu.CompilerParams(
            dimension_semantics=("arbitrary",)),
    )(pre)

    wd16 = W_dec.astype(jnp.bfloat16)
    blk_k = min(512, d_sae)
    n_k = d_sae // blk_k
    x_hat = pl.pallas_call(
        _dec_body,
        grid=(n_k,),
        in_specs=[
            pl.BlockSpec((rows, blk_k), lambda kk: (0, kk)),
            pl.BlockSpec((blk_k, d_model), lambda kk: (kk, 0)),
            pl.BlockSpec((1, d_model), lambda kk: (0, 0)),
        ],
        out_specs=pl.BlockSpec((rows, d_model), lambda kk: (0, 0)),
        out_shape=jax.ShapeDtypeStruct((rows, d_model), jnp.float32),
        compiler_params=pltpu.CompilerParams(
            dimension_semantics=("arbitrary",)),
    )(z, wd16, b_dec.reshape(1, d_model))

    return x_hat.reshape(b, s, d_model)
